# baseline (speedup 1.0000x reference)
import functools

import jax
import jax.numpy as jnp
from jax import lax
from jax.experimental import pallas as pl
from jax.experimental.pallas import tpu as pltpu
from jax.experimental.pallas import tpu_sc as plsc

N = 10000
E = 320000
FEAT = 128
HID = 48
NCLS = 16
K = 20
ALPHA = 0.1

NS = 16
NP = 640
NPAD = NS * NP
CHUNK = 128
NBUF = 4
NCH = 160
EP = NCH * CHUNK
EPAD = NS * EP


def _dense_body(x_ref, w1_ref, b1_ref, w2_ref, b2_ref, o_ref):
    h1 = jnp.dot(x_ref[...], w1_ref[...], preferred_element_type=jnp.float32)
    h1 = jnp.maximum(h1 + b1_ref[...], 0.0)
    o_ref[...] = (
        jnp.dot(h1, w2_ref[...], preferred_element_type=jnp.float32) + b2_ref[...]
    )


def _tc_dense(xp, w1t, b1, w2t, b2):
    return pl.pallas_call(
        _dense_body,
        grid=(NPAD // 1024,),
        in_specs=[
            pl.BlockSpec((1024, FEAT), lambda i: (i, 0)),
            pl.BlockSpec((FEAT, HID), lambda i: (0, 0)),
            pl.BlockSpec((1, HID), lambda i: (0, 0)),
            pl.BlockSpec((HID, NCLS), lambda i: (0, 0)),
            pl.BlockSpec((1, NCLS), lambda i: (0, 0)),
        ],
        out_specs=pl.BlockSpec((1024, NCLS), lambda i: (i, 0)),
        out_shape=jax.ShapeDtypeStruct((NPAD, NCLS), jnp.float32),
    )(xp, w1t, b1.reshape(1, HID), w2t, b2.reshape(1, NCLS))


def _lsm_body(z_ref, o_ref):
    z = z_ref[...]
    m = jnp.max(z, axis=1, keepdims=True)
    e = jnp.exp(z - m)
    s = jnp.sum(e, axis=1, keepdims=True)
    o_ref[...] = z - m - jnp.log(s)


def _tc_logsoftmax(z):
    return pl.pallas_call(
        _lsm_body,
        grid=(NPAD // 1024,),
        in_specs=[pl.BlockSpec((1024, NCLS), lambda i: (i, 0))],
        out_specs=pl.BlockSpec((1024, NCLS), lambda i: (i, 0)),
        out_shape=jax.ShapeDtypeStruct((NPAD, NCLS), jnp.float32),
    )(z)


_mesh = plsc.VectorSubcoreMesh(core_axis_name="c", subcore_axis_name="s",
                               num_cores=1, num_subcores=NS)


@functools.partial(
    pl.kernel,
    out_type=jax.ShapeDtypeStruct((NPAD, NCLS), jnp.float32),
    mesh=_mesh,
    compiler_params=pltpu.CompilerParams(
        needs_layout_passes=False, use_tc_tiling_on_sc=False
    ),
    scratch_types=[
        pltpu.VMEM_SHARED((NPAD // 16, 16), jnp.float32),
        pltpu.VMEM_SHARED((NPAD, NCLS), jnp.float32),
        pltpu.VMEM((NCH, CHUNK), jnp.int32),
        pltpu.VMEM((NCH, CHUNK), jnp.int32),
        pltpu.VMEM((NCH, CHUNK), jnp.float32),
        pltpu.VMEM((NPAD // 16, 16), jnp.float32),
        pltpu.VMEM((NP, NCLS), jnp.float32),
        pltpu.VMEM((NP, NCLS), jnp.float32),
        pltpu.VMEM((NP, NCLS), jnp.float32),
        pltpu.VMEM((NP // 16, 16), jnp.float32),
        pltpu.VMEM((NP // 16, 16), jnp.float32),
        [pltpu.VMEM((CHUNK, NCLS), jnp.float32) for _ in range(NBUF)],
        [pltpu.VMEM((CHUNK, NCLS), jnp.float32) for _ in range(NBUF)],
        [pltpu.SemaphoreType.DMA for _ in range(NBUF)],
        [pltpu.SemaphoreType.DMA for _ in range(NBUF)],
    ],
)
def _propagate(row_hbm, col_hbm, ew_hbm, h_hbm, z_hbm,
               dis_sh, agg_sh,
               row_loc, col_loc, wn_loc, disf, h_me, ua, uz,
               dis2, acc, gb, sb, sg, ss):
    sid = lax.axis_index("s")
    nbase = sid * NP
    nrow = sid * (NP // 16)
    zeros16 = jnp.zeros((16,), jnp.float32)

    pltpu.sync_copy(row_hbm.at[sid], row_loc)
    pltpu.sync_copy(col_hbm.at[sid], col_loc)
    pltpu.sync_copy(ew_hbm.at[sid], wn_loc)
    pltpu.sync_copy(h_hbm.at[pl.ds(nbase, NP)], h_me)

    def _zero_disf(r, _):
        disf[r, :] = zeros16
        return 0
    lax.fori_loop(0, NPAD // 16, _zero_disf, 0)

    def _deg(j, _):
        for g in range(CHUNK // 16):
            sl = pl.ds(g * 16, 16)
            c = col_loc[j, sl]
            plsc.addupdate_scatter(disf, [c >> 4, c & 15], wn_loc[j, sl])
        return 0
    lax.fori_loop(0, NCH, _deg, 0)
    pltpu.sync_copy(disf, z_hbm.at[pl.ds(nbase, NP)])
    plsc.subcore_barrier()

    ones16 = jnp.full((16,), 1.0, jnp.float32)
    NR = NP // 16

    def _init_acc(r, _):
        acc[r, :] = ones16
        return 0
    lax.fori_loop(0, NR, _init_acc, 0)
    for u in range(NS):
        pltpu.sync_copy(z_hbm.at[pl.ds(u * NP + nrow, NR)], ua.at[pl.ds(0, NR)])

        def _addp(r, _):
            acc[r, :] = acc[r, :] + ua[r, :]
            return 0
        lax.fori_loop(0, NR, _addp, 0)

    def _rsqrt(r, _):
        d = acc[r, :]
        bits = plsc.bitcast(d, jnp.int32)
        y = plsc.bitcast(jnp.int32(0x5F3759DF) - (bits >> 1), jnp.float32)
        for _ in range(3):
            y = y * (1.5 - 0.5 * d * y * y)
        ua[r, :] = y
        dis2[r, :] = y * y
        return 0
    lax.fori_loop(0, NR, _rsqrt, 0)
    pltpu.sync_copy(ua.at[pl.ds(0, NR)], dis_sh.at[pl.ds(nrow, NR)])
    plsc.subcore_barrier()

    pltpu.sync_copy(dis_sh, disf)

    def _norm(j, _):
        for g in range(CHUNK // 16):
            sl = pl.ds(g * 16, 16)
            r = row_loc[j, sl]
            c = col_loc[j, sl]
            a = plsc.load_gather(disf, [r >> 4, r & 15])
            b = plsc.load_gather(disf, [c >> 4, c & 15])
            wn_loc[j, sl] = a * wn_loc[j, sl] * b
        return 0
    lax.fori_loop(0, NCH, _norm, 0)

    def _zero_sb0(n, _):
        sb[0][n, :] = zeros16
        return 0
    lax.fori_loop(0, CHUNK, _zero_sb0, 0)

    def _cp_h(n, _):
        uz[n, :] = h_me[n, :]
        return 0
    lax.fori_loop(0, NP, _cp_h, 0)
    pltpu.sync_copy(uz, z_hbm.at[pl.ds(nbase, NP)])
    for q in range(NP // CHUNK):
        pltpu.sync_copy(sb[0], agg_sh.at[pl.ds(nbase + q * CHUNK, CHUNK)])
    plsc.subcore_barrier()

    def _gstart(j, b):
        pltpu.async_copy(z_hbm.at[row_loc.at[j]], gb[b], sg[b])

    def _gwait(j, b):
        pltpu.make_async_copy(z_hbm.at[row_loc.at[j]], gb[b], sg[b]).wait()

    def _sstart(j, b):
        pltpu.async_copy(sb[b], agg_sh.at[col_loc.at[j]], ss[b], add=True)

    def _swait(j, b):
        pltpu.make_async_copy(sb[b], agg_sh.at[col_loc.at[j]], ss[b]).wait()

    lane_consts = [jnp.full((16,), e, jnp.int32) for e in range(16)]

    def _scale(j, b):
        for g in range(CHUNK // 16):
            nv = wn_loc[j, pl.ds(g * 16, 16)]
            for e in range(16):
                idx = g * 16 + e
                bc = nv[lane_consts[e]]
                sb[b][idx, :] = gb[b][idx, :] * bc

    def _round(_, carry):
        for b in range(NBUF):
            _gstart(b, b)

        def _quad(jj, _c):
            j = NBUF * jj
            for b in range(NBUF):
                jb = j + b
                _gwait(jb, b)

                @pl.when(jj > 0)
                def _():
                    _swait(jb, b)
                _scale(jb, b)
                _sstart(jb, b)

                @pl.when(jj < NCH // NBUF - 1)
                def _():
                    _gstart(jb + NBUF, b)
            return 0
        lax.fori_loop(0, NCH // NBUF, _quad, 0)
        for b in range(NBUF):
            _swait(NCH - NBUF + b, b)
        plsc.subcore_barrier()

        pltpu.sync_copy(agg_sh.at[pl.ds(nbase, NP)], ua)

        def _upd(g, _c):
            d2v = dis2[g, :]
            for e in range(16):
                n = g * 16 + e
                zn = 0.9 * (ua[n, :] + d2v[e] * uz[n, :]) + 0.1 * h_me[n, :]
                uz[n, :] = zn
            return 0
        lax.fori_loop(0, NP // 16, _upd, 0)
        pltpu.sync_copy(uz, z_hbm.at[pl.ds(nbase, NP)])

        def _rezero_sb0(n, _c):
            sb[0][n, :] = zeros16
            return 0
        lax.fori_loop(0, CHUNK, _rezero_sb0, 0)
        for q in range(NP // CHUNK):
            pltpu.sync_copy(sb[0], agg_sh.at[pl.ds(nbase + q * CHUNK, CHUNK)])
        plsc.subcore_barrier()
        return carry
    lax.fori_loop(0, K, _round, 0)


def kernel(x, edge_index, edge_weight, W1, b1, W2, b2):
    row = edge_index[0].astype(jnp.int32)
    col = edge_index[1].astype(jnp.int32)
    ew = edge_weight.astype(jnp.float32)
    pad = EPAD - E
    row3 = jnp.pad(row, (0, pad)).reshape(NS, NCH, CHUNK)
    col3 = jnp.pad(col, (0, pad)).reshape(NS, NCH, CHUNK)
    ew3 = jnp.pad(ew, (0, pad)).reshape(NS, NCH, CHUNK)
    xp = jnp.pad(x, ((0, NPAD - N), (0, 0)))

    h = _tc_dense(xp, W1.T, b1, W2.T, b2)
    z = _propagate(row3, col3, ew3, h)
    return _tc_logsoftmax(z)[:N]


# speedup vs baseline: 1.0406x; 1.0406x over previous
import functools

import jax
import jax.numpy as jnp
from jax import lax
from jax.experimental import pallas as pl
from jax.experimental.pallas import tpu as pltpu
from jax.experimental.pallas import tpu_sc as plsc

N = 10000
E = 320000
FEAT = 128
HID = 48
NCLS = 16
K = 20
ALPHA = 0.1

NS = 16
NP = 640
NPAD = NS * NP
CHUNK = 128
NBUF = 3
NCH = 159
EP = NCH * CHUNK
EPAD = NS * EP


def _dense_body(x_ref, w1_ref, b1_ref, w2_ref, b2_ref, o_ref):
    h1 = jnp.dot(x_ref[...], w1_ref[...], preferred_element_type=jnp.float32)
    h1 = jnp.maximum(h1 + b1_ref[...], 0.0)
    o_ref[...] = (
        jnp.dot(h1, w2_ref[...], preferred_element_type=jnp.float32) + b2_ref[...]
    )


def _tc_dense(xp, w1t, b1, w2t, b2):
    return pl.pallas_call(
        _dense_body,
        grid=(NPAD // 1024,),
        in_specs=[
            pl.BlockSpec((1024, FEAT), lambda i: (i, 0)),
            pl.BlockSpec((FEAT, HID), lambda i: (0, 0)),
            pl.BlockSpec((1, HID), lambda i: (0, 0)),
            pl.BlockSpec((HID, NCLS), lambda i: (0, 0)),
            pl.BlockSpec((1, NCLS), lambda i: (0, 0)),
        ],
        out_specs=pl.BlockSpec((1024, NCLS), lambda i: (i, 0)),
        out_shape=jax.ShapeDtypeStruct((NPAD, NCLS), jnp.float32),
    )(xp, w1t, b1.reshape(1, HID), w2t, b2.reshape(1, NCLS))


def _lsm_body(z_ref, o_ref):
    z = z_ref[...]
    m = jnp.max(z, axis=1, keepdims=True)
    e = jnp.exp(z - m)
    s = jnp.sum(e, axis=1, keepdims=True)
    o_ref[...] = z - m - jnp.log(s)


def _tc_logsoftmax(z):
    return pl.pallas_call(
        _lsm_body,
        grid=(NPAD // 1024,),
        in_specs=[pl.BlockSpec((1024, NCLS), lambda i: (i, 0))],
        out_specs=pl.BlockSpec((1024, NCLS), lambda i: (i, 0)),
        out_shape=jax.ShapeDtypeStruct((NPAD, NCLS), jnp.float32),
    )(z)


_mesh = plsc.VectorSubcoreMesh(core_axis_name="c", subcore_axis_name="s",
                               num_cores=1, num_subcores=NS)


@functools.partial(
    pl.kernel,
    out_type=jax.ShapeDtypeStruct((NPAD, NCLS), jnp.float32),
    mesh=_mesh,
    compiler_params=pltpu.CompilerParams(
        needs_layout_passes=False, use_tc_tiling_on_sc=False
    ),
    scratch_types=[
        pltpu.VMEM_SHARED((NPAD // 16, 16), jnp.float32),
        pltpu.VMEM_SHARED((NPAD, NCLS), jnp.float32),
        pltpu.VMEM((NCH, CHUNK), jnp.int32),
        pltpu.VMEM((NCH, CHUNK), jnp.int32),
        pltpu.VMEM((NCH, CHUNK), jnp.float32),
        pltpu.VMEM((NPAD // 16, 16), jnp.float32),
        pltpu.VMEM((NP, NCLS), jnp.float32),
        pltpu.VMEM((NP, NCLS), jnp.float32),
        pltpu.VMEM((NP, NCLS), jnp.float32),
        pltpu.VMEM((CHUNK, NCLS), jnp.float32),
        pltpu.VMEM((NP // 16, 16), jnp.float32),
        pltpu.VMEM((NP // 16, 16), jnp.float32),
        [pltpu.VMEM((CHUNK, NCLS), jnp.float32) for _ in range(NBUF)],
        [pltpu.VMEM((CHUNK, NCLS), jnp.float32) for _ in range(NBUF)],
        [pltpu.SemaphoreType.DMA for _ in range(NBUF)],
        [pltpu.SemaphoreType.DMA for _ in range(NBUF)],
    ],
)
def _propagate(row_hbm, col_hbm, ew_hbm, h_hbm, z_hbm,
               dis_sh, agg_sh,
               row_loc, col_loc, wn_loc, disf, h_me, ua, uz, zc,
               dis2, acc, gb, sb, sg, ss):
    sid = lax.axis_index("s")
    nbase = sid * NP
    nrow = sid * (NP // 16)
    zeros16 = jnp.zeros((16,), jnp.float32)

    pltpu.sync_copy(row_hbm.at[sid], row_loc)
    pltpu.sync_copy(col_hbm.at[sid], col_loc)
    pltpu.sync_copy(ew_hbm.at[sid], wn_loc)
    pltpu.sync_copy(h_hbm.at[pl.ds(nbase, NP)], h_me)

    def _zero_disf(r, _):
        disf[r, :] = zeros16
        return 0
    lax.fori_loop(0, NPAD // 16, _zero_disf, 0)

    def _deg(j, _):
        for g in range(CHUNK // 16):
            sl = pl.ds(g * 16, 16)
            c = col_loc[j, sl]
            plsc.addupdate_scatter(disf, [c >> 4, c & 15], wn_loc[j, sl])
        return 0
    lax.fori_loop(0, NCH, _deg, 0)
    pltpu.sync_copy(disf, z_hbm.at[pl.ds(nbase, NP)])
    plsc.subcore_barrier()

    ones16 = jnp.full((16,), 1.0, jnp.float32)
    NR = NP // 16

    def _init_acc(r, _):
        acc[r, :] = ones16
        return 0
    lax.fori_loop(0, NR, _init_acc, 0)
    for u in range(NS):
        pltpu.sync_copy(z_hbm.at[pl.ds(u * NP + nrow, NR)], ua.at[pl.ds(0, NR)])

        def _addp(r, _):
            acc[r, :] = acc[r, :] + ua[r, :]
            return 0
        lax.fori_loop(0, NR, _addp, 0)

    def _rsqrt(r, _):
        d = acc[r, :]
        bits = plsc.bitcast(d, jnp.int32)
        y = plsc.bitcast(jnp.int32(0x5F3759DF) - (bits >> 1), jnp.float32)
        for _ in range(3):
            y = y * (1.5 - 0.5 * d * y * y)
        ua[r, :] = y
        dis2[r, :] = y * y
        return 0
    lax.fori_loop(0, NR, _rsqrt, 0)
    pltpu.sync_copy(ua.at[pl.ds(0, NR)], dis_sh.at[pl.ds(nrow, NR)])
    plsc.subcore_barrier()

    pltpu.sync_copy(dis_sh, disf)

    def _norm(j, _):
        for g in range(CHUNK // 16):
            sl = pl.ds(g * 16, 16)
            r = row_loc[j, sl]
            c = col_loc[j, sl]
            a = plsc.load_gather(disf, [r >> 4, r & 15])
            b = plsc.load_gather(disf, [c >> 4, c & 15])
            wn_loc[j, sl] = a * wn_loc[j, sl] * b
        return 0
    lax.fori_loop(0, NCH, _norm, 0)

    def _zero_zc(n, _):
        zc[n, :] = zeros16
        return 0
    lax.fori_loop(0, CHUNK, _zero_zc, 0)

    def _cp_h(n, _):
        uz[n, :] = h_me[n, :]
        return 0
    lax.fori_loop(0, NP, _cp_h, 0)
    pltpu.sync_copy(uz, z_hbm.at[pl.ds(nbase, NP)])
    for q in range(NP // CHUNK):
        pltpu.sync_copy(zc, agg_sh.at[pl.ds(nbase + q * CHUNK, CHUNK)])
    plsc.subcore_barrier()

    def _gstart(j, b):
        pltpu.async_copy(z_hbm.at[row_loc.at[j]], gb[b], sg[b])

    def _gwait(j, b):
        pltpu.make_async_copy(z_hbm.at[row_loc.at[j]], gb[b], sg[b]).wait()

    def _sstart(j, b):
        pltpu.async_copy(sb[b], agg_sh.at[col_loc.at[j]], ss[b], add=True)

    def _swait(j, b):
        pltpu.make_async_copy(sb[b], agg_sh.at[col_loc.at[j]], ss[b]).wait()

    lane_consts = [jnp.full((16,), e, jnp.int32) for e in range(16)]

    def _scale(j, b):
        for g in range(CHUNK // 16):
            nv = wn_loc[j, pl.ds(g * 16, 16)]
            for e in range(16):
                idx = g * 16 + e
                bc = nv[lane_consts[e]]
                sb[b][idx, :] = gb[b][idx, :] * bc

    def _round(_, carry):
        for b in range(NBUF):
            _gstart(b, b)

        def _quad(jj, _c):
            j = NBUF * jj
            for b in range(NBUF):
                jb = j + b
                _gwait(jb, b)

                @pl.when(jj > 0)
                def _():
                    _swait(jb, b)
                _scale(jb, b)
                _sstart(jb, b)

                @pl.when(jj < NCH // NBUF - 1)
                def _():
                    _gstart(jb + NBUF, b)
            return 0
        lax.fori_loop(0, NCH // NBUF, _quad, 0)
        for b in range(NBUF):
            _swait(NCH - NBUF + b, b)
        plsc.subcore_barrier()

        pltpu.sync_copy(agg_sh.at[pl.ds(nbase, NP)], ua)

        def _upd(g, _c):
            d2v = dis2[g, :]
            for e in range(16):
                n = g * 16 + e
                zn = 0.9 * (ua[n, :] + d2v[e] * uz[n, :]) + 0.1 * h_me[n, :]
                uz[n, :] = zn
            return 0
        lax.fori_loop(0, NP // 16, _upd, 0)
        pltpu.sync_copy(uz, z_hbm.at[pl.ds(nbase, NP)])
        for q in range(NP // CHUNK):
            pltpu.sync_copy(zc, agg_sh.at[pl.ds(nbase + q * CHUNK, CHUNK)])
        plsc.subcore_barrier()
        return carry
    lax.fori_loop(0, K, _round, 0)


def kernel(x, edge_index, edge_weight, W1, b1, W2, b2):
    row = edge_index[0].astype(jnp.int32)
    col = edge_index[1].astype(jnp.int32)
    ew = edge_weight.astype(jnp.float32)
    pad = EPAD - E
    row3 = jnp.pad(row, (0, pad)).reshape(NS, NCH, CHUNK)
    col3 = jnp.pad(col, (0, pad)).reshape(NS, NCH, CHUNK)
    ew3 = jnp.pad(ew, (0, pad)).reshape(NS, NCH, CHUNK)
    xp = jnp.pad(x, ((0, NPAD - N), (0, 0)))

    h = _tc_dense(xp, W1.T, b1, W2.T, b2)
    z = _propagate(row3, col3, ew3, h)
    return _tc_logsoftmax(z)[:N]
